# baseline (speedup 1.0000x reference)
import functools

import numpy as np
import jax
import jax.numpy as jnp
from jax import lax
from jax.experimental import pallas as pl
from jax.experimental.pallas import tpu as pltpu
from jax.experimental.pallas import tpu_sc as plsc

_C = 192
_H = 384
_PT = 22
_N = _H * _H
_CB = 16
_YB = 16


def _interp_weight_mat(h, n_in):
    i = np.arange(h, dtype=np.float64)
    s = (i + 0.5) * (n_in / h) - 0.5
    a = np.arange(n_in, dtype=np.float64)
    w = np.maximum(0.0, 1.0 - np.abs(s[None, :] - a[:, None]))
    w = w / w.sum(axis=0, keepdims=True)
    return w.astype(np.float32)


def _sc_gather_body(widx_hbm, ptab_hbm, out_hbm,
                    widx_v, rows_v, out_v, sem):
    info = plsc.get_sparse_core_info()
    wid = lax.axis_index("s") * info.num_cores + lax.axis_index("c")

    @pl.when(wid < _C // 16)
    def _():
        base = wid * 16
        pltpu.sync_copy(widx_hbm, widx_v)
        pltpu.async_copy(ptab_hbm.at[widx_v], rows_v, sem).wait()
        for i in range(18):
            out_v[i, :] = rows_v[i, pl.ds(base, 16)]
        zero = jnp.zeros((16,), jnp.float32)
        for col in range(18, 32):
            out_v[col, :] = zero
        pltpu.sync_copy(out_v, out_hbm.at[wid])


_sc_gather = functools.partial(
    pl.kernel,
    out_type=jax.ShapeDtypeStruct((_C // 16, 32, 16), jnp.float32),
    mesh=plsc.VectorSubcoreMesh(core_axis_name="c", subcore_axis_name="s"),
    scratch_types=[
        pltpu.VMEM((32,), jnp.int32),
        pltpu.VMEM((32, 256), jnp.float32),
        pltpu.VMEM((32, 16), jnp.float32),
        pltpu.SemaphoreType.DMA,
    ],
)(_sc_gather_body)


def _norm_body(gw_ref, w_ref, b_ref, scal_ref, x_ref, wy_ref,
               wx_ref, o_ref, st_ref, sums_ref):
    p = pl.program_id(0)
    c = pl.program_id(1)

    @pl.when(p == 0)
    def _stats():
        xb = x_ref[0]
        sums_ref[pl.ds(c * _CB, _CB), 0:1] = jnp.sum(
            xb, axis=(1, 2))[:, None]
        sums_ref[pl.ds(c * _CB, _CB), 1:2] = jnp.sum(
            xb * xb, axis=(1, 2))[:, None]

    @pl.when((p == 1) & (c == 0))
    def _fold():
        wm = gw_ref[:, 0:9]
        ws = gw_ref[:, 9:18]
        s1 = sums_ref[:, 0:1]
        s2 = sums_ref[:, 1:2]
        n = jnp.float32(_N)
        pm = s1 / n
        pv = (s2 - s1 * s1 / n) / (n - 1.0)
        ps = jnp.sqrt(pv)
        um = scal_ref[0:1, 0:9]
        wm = wm * (1.0 - um) + pm * um
        ws = ws * (1.0 - um) + ps * um
        cm = wm[:, 4:5]
        cs = ws[:, 4:5]
        wm = jnp.where(wm == 0.0, cm, wm)
        ws = jnp.where(ws == 0.0, cs, ws)
        af = scal_ref[0:1, 9:10]
        pf = scal_ref[0:1, 10:11]
        w = w_ref[...]
        b = b_ref[...]
        wm = wm * af
        ws = ws * af + (1.0 - af)
        wr = w * af + (1.0 - af)
        br = b * af
        ainv = w / ps
        a_pre = ainv * pf + (1.0 - pf)
        b_pre = (b - pm * ainv) * pf
        st_ref[:, 0:9] = wm
        st_ref[:, 9:18] = ws
        st_ref[:, 18:19] = a_pre
        st_ref[:, 19:20] = b_pre
        st_ref[:, 20:21] = wr
        st_ref[:, 21:22] = br

    sl = st_ref[pl.ds(c * _CB, _CB), :]

    @pl.when(p == 1)
    def _real():
        xb = x_ref[0]
        wyb = wy_ref[...]
        for cc in range(_CB):
            rows_m = []
            rows_s = []
            for a in range(3):
                row_m = None
                row_s = None
                for bb in range(3):
                    wxv = wx_ref[bb:bb + 1, :]
                    m = sl[cc:cc + 1, 3 * a + bb:3 * a + bb + 1]
                    s = sl[cc:cc + 1, 9 + 3 * a + bb:10 + 3 * a + bb]
                    row_m = m * wxv if row_m is None else row_m + m * wxv
                    row_s = s * wxv if row_s is None else row_s + s * wxv
                rows_m.append(row_m)
                rows_s.append(row_s)
            rm = jnp.concatenate(rows_m, axis=0)
            rs = jnp.concatenate(rows_s, axis=0)
            mean = jnp.dot(wyb, rm, preferred_element_type=jnp.float32)
            std = jnp.dot(wyb, rs, preferred_element_type=jnp.float32)
            wr = sl[cc:cc + 1, 20:21]
            br = sl[cc:cc + 1, 21:22]
            o_ref[0, cc] = (xb[cc] - mean) / std * wr + br

    @pl.when(p == 2)
    def _pre():
        xb = x_ref[...]
        a_pre = sl[:, 18:19].reshape(1, _CB, 1, 1)
        b_pre = sl[:, 19:20].reshape(1, _CB, 1, 1)
        o_ref[...] = xb * a_pre + b_pre


def kernel(x, weight, bias, mean_table, std_table, padded_mean_table,
           padded_std_table, y_anchor, x_anchor, padding, pre_y_anchor,
           pre_x_anchor):
    pad_static = (padded_mean_table.shape[2] - mean_table.shape[0]) // 2
    top = y_anchor + padding - pad_static
    left = x_anchor + padding - pad_static
    ry = pre_y_anchor + 1 - top
    rx = pre_x_anchor + 1 - left
    in_win = ((pre_y_anchor != -1) & (ry >= 0) & (ry < 3)
              & (rx >= 0) & (rx < 3))
    k_pre = jnp.where(in_win, ry * 3 + rx, 100)
    upd = (jnp.arange(9) == k_pre).astype(jnp.float32)
    scal = jnp.concatenate([
        upd,
        jnp.asarray(y_anchor != -1, jnp.float32).reshape(1),
        jnp.asarray(pre_y_anchor != -1, jnp.float32).reshape(1),
        jnp.zeros((5,), jnp.float32),
    ]).reshape(1, 16)
    rows = [(top + dy) * _PT + (left + dx)
            for dy in range(3) for dx in range(3)]
    widx = jnp.clip(jnp.stack(rows).astype(jnp.int32), 0, _PT * _PT - 1)
    widx = jnp.concatenate([widx, widx + _PT * _PT,
                            jnp.zeros((14,), jnp.int32)])

    ptab = jnp.pad(
        jnp.concatenate([padded_mean_table[0].reshape(_C, _PT * _PT).T,
                         padded_std_table[0].reshape(_C, _PT * _PT).T],
                        axis=0),
        ((0, 0), (0, 256 - _C)))
    wvec = weight.reshape(_C, 1)
    bvec = bias.reshape(_C, 1)

    gw3 = _sc_gather(widx, ptab)
    gw = gw3.transpose(0, 2, 1).reshape(_C, 32)

    wy = jnp.asarray(_interp_weight_mat(_H, 3).T)
    wx = jnp.asarray(_interp_weight_mat(_H, 3))

    def _x_idx(p, c):
        return (jnp.where(p == 1, 0, 1), c, 0, 0)

    def _o_idx(p, c):
        zero = jnp.int32(0)
        return (jnp.where(p == 2, 1, 0),
                jnp.where(p == 0, zero, c), 0, 0)

    out = pl.pallas_call(
        _norm_body,
        grid=(3, _C // _CB),
        in_specs=[
            pl.BlockSpec((_C, 32), lambda p, c: (0, 0)),
            pl.BlockSpec((_C, 1), lambda p, c: (0, 0)),
            pl.BlockSpec((_C, 1), lambda p, c: (0, 0)),
            pl.BlockSpec((1, 16), lambda p, c: (0, 0)),
            pl.BlockSpec((1, _CB, _H, _H), _x_idx),
            pl.BlockSpec((_H, 3), lambda p, c: (0, 0)),
            pl.BlockSpec((3, _H), lambda p, c: (0, 0)),
        ],
        out_specs=pl.BlockSpec((1, _CB, _H, _H), _o_idx),
        out_shape=jax.ShapeDtypeStruct((2, _C, _H, _H), jnp.float32),
        scratch_shapes=[pltpu.VMEM((_C, 32), jnp.float32),
                        pltpu.VMEM((_C, 8), jnp.float32)],
    )(gw, wvec, bvec, scal, x, wy, wx)

    return out


# speedup vs baseline: 1.0056x; 1.0056x over previous
import functools

import numpy as np
import jax
import jax.numpy as jnp
from jax import lax
from jax.experimental import pallas as pl
from jax.experimental.pallas import tpu as pltpu
from jax.experimental.pallas import tpu_sc as plsc

_C = 192
_H = 384
_PT = 22
_N = _H * _H
_CB = 16
_YB = 16


def _interp_weight_mat(h, n_in):
    i = np.arange(h, dtype=np.float64)
    s = (i + 0.5) * (n_in / h) - 0.5
    a = np.arange(n_in, dtype=np.float64)
    w = np.maximum(0.0, 1.0 - np.abs(s[None, :] - a[:, None]))
    w = w / w.sum(axis=0, keepdims=True)
    return w.astype(np.float32)


def _sc_gather_body(widx_hbm, ptm_hbm, pts_hbm, out_hbm,
                    widx_v, mrows_v, srows_v, out_v, sem):
    info = plsc.get_sparse_core_info()
    wid = lax.axis_index("s") * info.num_cores + lax.axis_index("c")

    @pl.when(wid < _C // 16)
    def _():
        base = wid * 16
        pltpu.sync_copy(widx_hbm, widx_v)
        pltpu.async_copy(ptm_hbm.at[widx_v], mrows_v, sem).wait()
        pltpu.async_copy(pts_hbm.at[widx_v], srows_v, sem).wait()
        for i in range(9):
            out_v[i, :] = mrows_v[i, pl.ds(base, 16)]
            out_v[9 + i, :] = srows_v[i, pl.ds(base, 16)]
        zero = jnp.zeros((16,), jnp.float32)
        for col in range(18, 32):
            out_v[col, :] = zero
        pltpu.sync_copy(out_v, out_hbm.at[wid])


_sc_gather = functools.partial(
    pl.kernel,
    out_type=jax.ShapeDtypeStruct((_C // 16, 32, 16), jnp.float32),
    mesh=plsc.VectorSubcoreMesh(core_axis_name="c", subcore_axis_name="s"),
    scratch_types=[
        pltpu.VMEM((16,), jnp.int32),
        pltpu.VMEM((16, 256), jnp.float32),
        pltpu.VMEM((16, 256), jnp.float32),
        pltpu.VMEM((32, 16), jnp.float32),
        pltpu.SemaphoreType.DMA,
    ],
)(_sc_gather_body)


def _norm_body(gw_ref, w_ref, b_ref, scal_ref, x_ref, wy_ref,
               wx_ref, o_ref, st_ref, sums_ref):
    p = pl.program_id(0)
    c = pl.program_id(1)

    @pl.when(p == 0)
    def _stats():
        xb = x_ref[0]
        sums_ref[pl.ds(c * _CB, _CB), 0:1] = jnp.sum(
            xb, axis=(1, 2))[:, None]
        sums_ref[pl.ds(c * _CB, _CB), 1:2] = jnp.sum(
            xb * xb, axis=(1, 2))[:, None]

    @pl.when((p == 1) & (c == 0))
    def _fold():
        wm = gw_ref[:, 0:9]
        ws = gw_ref[:, 9:18]
        s1 = sums_ref[:, 0:1]
        s2 = sums_ref[:, 1:2]
        n = jnp.float32(_N)
        pm = s1 / n
        pv = (s2 - s1 * s1 / n) / (n - 1.0)
        ps = jnp.sqrt(pv)
        um = scal_ref[0:1, 0:9]
        wm = wm * (1.0 - um) + pm * um
        ws = ws * (1.0 - um) + ps * um
        cm = wm[:, 4:5]
        cs = ws[:, 4:5]
        wm = jnp.where(wm == 0.0, cm, wm)
        ws = jnp.where(ws == 0.0, cs, ws)
        af = scal_ref[0:1, 9:10]
        pf = scal_ref[0:1, 10:11]
        w = w_ref[...]
        b = b_ref[...]
        wm = wm * af
        ws = ws * af + (1.0 - af)
        wr = w * af + (1.0 - af)
        br = b * af
        ainv = w / ps
        a_pre = ainv * pf + (1.0 - pf)
        b_pre = (b - pm * ainv) * pf
        st_ref[:, 0:9] = wm
        st_ref[:, 9:18] = ws
        st_ref[:, 18:19] = a_pre
        st_ref[:, 19:20] = b_pre
        st_ref[:, 20:21] = wr
        st_ref[:, 21:22] = br

    sl = st_ref[pl.ds(c * _CB, _CB), :]

    @pl.when(p == 1)
    def _real():
        xb = x_ref[0]
        wyb = wy_ref[...]
        for cc in range(_CB):
            rows_m = []
            rows_s = []
            for a in range(3):
                row_m = None
                row_s = None
                for bb in range(3):
                    wxv = wx_ref[bb:bb + 1, :]
                    m = sl[cc:cc + 1, 3 * a + bb:3 * a + bb + 1]
                    s = sl[cc:cc + 1, 9 + 3 * a + bb:10 + 3 * a + bb]
                    row_m = m * wxv if row_m is None else row_m + m * wxv
                    row_s = s * wxv if row_s is None else row_s + s * wxv
                rows_m.append(row_m)
                rows_s.append(row_s)
            rm = jnp.concatenate(rows_m, axis=0)
            rs = jnp.concatenate(rows_s, axis=0)
            mean = jnp.dot(wyb, rm, preferred_element_type=jnp.float32)
            std = jnp.dot(wyb, rs, preferred_element_type=jnp.float32)
            wr = sl[cc:cc + 1, 20:21]
            br = sl[cc:cc + 1, 21:22]
            o_ref[0, cc] = (xb[cc] - mean) / std * wr + br

    @pl.when(p == 2)
    def _pre():
        xb = x_ref[...]
        a_pre = sl[:, 18:19].reshape(1, _CB, 1, 1)
        b_pre = sl[:, 19:20].reshape(1, _CB, 1, 1)
        o_ref[...] = xb * a_pre + b_pre


def kernel(x, weight, bias, mean_table, std_table, padded_mean_table,
           padded_std_table, y_anchor, x_anchor, padding, pre_y_anchor,
           pre_x_anchor):
    pad_static = (padded_mean_table.shape[2] - mean_table.shape[0]) // 2
    top = y_anchor + padding - pad_static
    left = x_anchor + padding - pad_static
    ry = pre_y_anchor + 1 - top
    rx = pre_x_anchor + 1 - left
    in_win = ((pre_y_anchor != -1) & (ry >= 0) & (ry < 3)
              & (rx >= 0) & (rx < 3))
    k_pre = jnp.where(in_win, ry * 3 + rx, 100)
    upd = (jnp.arange(9) == k_pre).astype(jnp.float32)
    scal = jnp.concatenate([
        upd,
        jnp.asarray(y_anchor != -1, jnp.float32).reshape(1),
        jnp.asarray(pre_y_anchor != -1, jnp.float32).reshape(1),
        jnp.zeros((5,), jnp.float32),
    ]).reshape(1, 16)
    rows = [(top + dy) * _PT + (left + dx)
            for dy in range(3) for dx in range(3)]
    widx = jnp.clip(jnp.stack(rows).astype(jnp.int32), 0, _PT * _PT - 1)
    widx = jnp.concatenate([widx, jnp.zeros((7,), jnp.int32)])

    ptm = jnp.pad(padded_mean_table[0].reshape(_C, _PT * _PT).T,
                  ((0, 0), (0, 256 - _C)))
    pts = jnp.pad(padded_std_table[0].reshape(_C, _PT * _PT).T,
                  ((0, 0), (0, 256 - _C)))
    wvec = weight.reshape(_C, 1)
    bvec = bias.reshape(_C, 1)

    gw3 = _sc_gather(widx, ptm, pts)
    gw = gw3.transpose(0, 2, 1).reshape(_C, 32)

    wy = jnp.asarray(_interp_weight_mat(_H, 3).T)
    wx = jnp.asarray(_interp_weight_mat(_H, 3))

    def _x_idx(p, c):
        return (jnp.where(p == 1, 0, 1), c, 0, 0)

    def _o_idx(p, c):
        zero = jnp.int32(0)
        return (jnp.where(p == 2, 1, 0),
                jnp.where(p == 0, zero, c), 0, 0)

    out = pl.pallas_call(
        _norm_body,
        grid=(3, _C // _CB),
        in_specs=[
            pl.BlockSpec((_C, 32), lambda p, c: (0, 0)),
            pl.BlockSpec((_C, 1), lambda p, c: (0, 0)),
            pl.BlockSpec((_C, 1), lambda p, c: (0, 0)),
            pl.BlockSpec((1, 16), lambda p, c: (0, 0)),
            pl.BlockSpec((1, _CB, _H, _H), _x_idx),
            pl.BlockSpec((_H, 3), lambda p, c: (0, 0)),
            pl.BlockSpec((3, _H), lambda p, c: (0, 0)),
        ],
        out_specs=pl.BlockSpec((1, _CB, _H, _H), _o_idx),
        out_shape=jax.ShapeDtypeStruct((2, _C, _H, _H), jnp.float32),
        scratch_shapes=[pltpu.VMEM((_C, 32), jnp.float32),
                        pltpu.VMEM((_C, 8), jnp.float32)],
    )(gw, wvec, bvec, scal, x, wy, wx)

    return out


# speedup vs baseline: 1.0066x; 1.0010x over previous
import functools

import numpy as np
import jax
import jax.numpy as jnp
from jax import lax
from jax.experimental import pallas as pl
from jax.experimental.pallas import tpu as pltpu
from jax.experimental.pallas import tpu_sc as plsc

_C = 192
_H = 384
_PT = 22
_N = _H * _H
_CB = 16


def _interp_weight_mat(h, n_in):
    i = np.arange(h, dtype=np.float64)
    s = (i + 0.5) * (n_in / h) - 0.5
    a = np.arange(n_in, dtype=np.float64)
    w = np.maximum(0.0, 1.0 - np.abs(s[None, :] - a[:, None]))
    w = w / w.sum(axis=0, keepdims=True)
    return w.astype(np.float32)


def _sc_gather_body(widx_hbm, ptm_hbm, pts_hbm, out_hbm,
                    widx_v, mrows_v, srows_v, out_v, sem):
    info = plsc.get_sparse_core_info()
    wid = lax.axis_index("s") * info.num_cores + lax.axis_index("c")

    @pl.when(wid < _C // 16)
    def _():
        base = wid * 16
        pltpu.sync_copy(widx_hbm, widx_v)
        pltpu.async_copy(ptm_hbm.at[widx_v], mrows_v, sem).wait()
        pltpu.async_copy(pts_hbm.at[widx_v], srows_v, sem).wait()
        for i in range(9):
            out_v[i, :] = mrows_v[i, pl.ds(base, 16)]
            out_v[9 + i, :] = srows_v[i, pl.ds(base, 16)]
        zero = jnp.zeros((16,), jnp.float32)
        for col in range(18, 32):
            out_v[col, :] = zero
        pltpu.sync_copy(out_v, out_hbm.at[wid])


_sc_gather = functools.partial(
    pl.kernel,
    out_type=jax.ShapeDtypeStruct((_C // 16, 32, 16), jnp.float32),
    mesh=plsc.VectorSubcoreMesh(core_axis_name="c", subcore_axis_name="s"),
    scratch_types=[
        pltpu.VMEM((16,), jnp.int32),
        pltpu.VMEM((16, 256), jnp.float32),
        pltpu.VMEM((16, 256), jnp.float32),
        pltpu.VMEM((32, 16), jnp.float32),
        pltpu.SemaphoreType.DMA,
    ],
)(_sc_gather_body)


def _norm_body(gw_ref, w_ref, b_ref, scal_ref, x_ref, wy_ref,
               wx_ref, o_ref, st_ref, sums_ref):
    p = pl.program_id(0)
    c = pl.program_id(1)

    @pl.when(p == 0)
    def _stats():
        xb = x_ref[0]
        sums_ref[pl.ds(c * _CB, _CB), 0:1] = jnp.sum(
            xb, axis=(1, 2))[:, None]
        sums_ref[pl.ds(c * _CB, _CB), 1:2] = jnp.sum(
            xb * xb, axis=(1, 2))[:, None]

    @pl.when((p == 1) & (c == 0))
    def _fold():
        wm = gw_ref[:, 0:9]
        ws = gw_ref[:, 9:18]
        s1 = sums_ref[:, 0:1]
        s2 = sums_ref[:, 1:2]
        n = jnp.float32(_N)
        pm = s1 / n
        pv = (s2 - s1 * s1 / n) / (n - 1.0)
        ps = jnp.sqrt(pv)
        um = scal_ref[0:1, 0:9]
        wm = wm * (1.0 - um) + pm * um
        ws = ws * (1.0 - um) + ps * um
        cm = wm[:, 4:5]
        cs = ws[:, 4:5]
        wm = jnp.where(wm == 0.0, cm, wm)
        ws = jnp.where(ws == 0.0, cs, ws)
        af = scal_ref[0:1, 9:10]
        pf = scal_ref[0:1, 10:11]
        w = w_ref[...]
        b = b_ref[...]
        wm = wm * af
        ws = ws * af + (1.0 - af)
        wr = w * af + (1.0 - af)
        br = b * af
        ainv = w / ps
        a_pre = ainv * pf + (1.0 - pf)
        b_pre = (b - pm * ainv) * pf
        st_ref[:, 0:9] = wm
        st_ref[:, 9:18] = ws
        st_ref[:, 18:19] = a_pre
        st_ref[:, 19:20] = b_pre
        st_ref[:, 20:21] = wr
        st_ref[:, 21:22] = br

    sl = st_ref[pl.ds(c * _CB, _CB), :]

    @pl.when(p == 1)
    def _real():
        xb = x_ref[0]
        wyb = wy_ref[...]
        for cc in range(_CB):
            rows_m = []
            rows_s = []
            for a in range(3):
                row_m = None
                row_s = None
                for bb in range(3):
                    wxv = wx_ref[bb:bb + 1, :]
                    m = sl[cc:cc + 1, 3 * a + bb:3 * a + bb + 1]
                    s = sl[cc:cc + 1, 9 + 3 * a + bb:10 + 3 * a + bb]
                    row_m = m * wxv if row_m is None else row_m + m * wxv
                    row_s = s * wxv if row_s is None else row_s + s * wxv
                rows_m.append(row_m)
                rows_s.append(row_s)
            rm = jnp.concatenate(rows_m, axis=0)
            rs = jnp.concatenate(rows_s, axis=0)
            mean = jnp.dot(wyb, rm, preferred_element_type=jnp.float32)
            std = jnp.dot(wyb, rs, preferred_element_type=jnp.float32)
            wr = sl[cc:cc + 1, 20:21]
            br = sl[cc:cc + 1, 21:22]
            o_ref[0, cc] = (xb[cc] - mean) / std * wr + br

    @pl.when(p == 2)
    def _pre():
        xb = x_ref[...]
        a_pre = sl[:, 18:19].reshape(1, _CB, 1, 1)
        b_pre = sl[:, 19:20].reshape(1, _CB, 1, 1)
        o_ref[...] = xb * a_pre + b_pre


def kernel(x, weight, bias, mean_table, std_table, padded_mean_table,
           padded_std_table, y_anchor, x_anchor, padding, pre_y_anchor,
           pre_x_anchor):
    pad_static = (padded_mean_table.shape[2] - mean_table.shape[0]) // 2
    top = y_anchor + padding - pad_static
    left = x_anchor + padding - pad_static
    ry = pre_y_anchor + 1 - top
    rx = pre_x_anchor + 1 - left
    in_win = ((pre_y_anchor != -1) & (ry >= 0) & (ry < 3)
              & (rx >= 0) & (rx < 3))
    k_pre = jnp.where(in_win, ry * 3 + rx, 100)
    upd = (jnp.arange(9) == k_pre).astype(jnp.float32)
    scal = jnp.concatenate([
        upd,
        jnp.asarray(y_anchor != -1, jnp.float32).reshape(1),
        jnp.asarray(pre_y_anchor != -1, jnp.float32).reshape(1),
        jnp.zeros((5,), jnp.float32),
    ]).reshape(1, 16)
    rows = [(top + dy) * _PT + (left + dx)
            for dy in range(3) for dx in range(3)]
    widx = jnp.clip(jnp.stack(rows).astype(jnp.int32), 0, _PT * _PT - 1)
    widx = jnp.concatenate([widx, jnp.zeros((7,), jnp.int32)])

    ptm = jnp.pad(padded_mean_table[0].reshape(_C, _PT * _PT).T,
                  ((0, 0), (0, 256 - _C)))
    pts = jnp.pad(padded_std_table[0].reshape(_C, _PT * _PT).T,
                  ((0, 0), (0, 256 - _C)))
    wvec = weight.reshape(_C, 1)
    bvec = bias.reshape(_C, 1)

    gw3 = _sc_gather(widx, ptm, pts)
    gw = gw3.transpose(0, 2, 1).reshape(_C, 32)

    wy = jnp.asarray(_interp_weight_mat(_H, 3).T)
    wx = jnp.asarray(_interp_weight_mat(_H, 3))

    def _x_idx(p, c):
        return (jnp.where(p == 1, 0, 1), c, 0, 0)

    def _o_idx(p, c):
        zero = jnp.int32(0)
        return (jnp.where(p == 2, 1, 0),
                jnp.where(p == 0, zero, c), 0, 0)

    out = pl.pallas_call(
        _norm_body,
        grid=(3, _C // _CB),
        in_specs=[
            pl.BlockSpec((_C, 32), lambda p, c: (0, 0)),
            pl.BlockSpec((_C, 1), lambda p, c: (0, 0)),
            pl.BlockSpec((_C, 1), lambda p, c: (0, 0)),
            pl.BlockSpec((1, 16), lambda p, c: (0, 0)),
            pl.BlockSpec((1, _CB, _H, _H), _x_idx),
            pl.BlockSpec((_H, 3), lambda p, c: (0, 0)),
            pl.BlockSpec((3, _H), lambda p, c: (0, 0)),
        ],
        out_specs=pl.BlockSpec((1, _CB, _H, _H), _o_idx),
        out_shape=jax.ShapeDtypeStruct((2, _C, _H, _H), jnp.float32),
        scratch_shapes=[pltpu.VMEM((_C, 32), jnp.float32),
                        pltpu.VMEM((_C, 8), jnp.float32)],
    )(gw, wvec, bvec, scal, x, wy, wx)

    return out
